# Initial kernel scaffold; baseline (speedup 1.0000x reference)
import jax
import jax.numpy as jnp
from jax.experimental import pallas as pl


def kernel(obs_traj, hyperedge_indices, W_enc, b_enc, theta0, theta1, gamma0, beta0, gamma1, beta1, W_dec, b_dec):
    raise NotImplementedError("write your pallas kernel here")


# speedup vs baseline: 10.8470x; 10.8470x over previous
import functools

import jax
import jax.numpy as jnp
from jax import lax
from jax.experimental import pallas as pl
from jax.experimental.pallas import tpu as pltpu
from jax.experimental.pallas import tpu_sc as plsc

B, N, T, D_IN = 4, 10000, 8, 2
E = 160000
H = 128
PRED = 12
M = 10000
S = 10000
EPS = 1e-5

NC, NS = 2, 16
CHUNK = 40
PER_TILE = E // (NC * NS)
NCHUNK = PER_TILE // CHUNK
SP = 10240
RPT = SP // NS
ZROWS = 32

_mesh = plsc.VectorSubcoreMesh(core_axis_name="c", subcore_axis_name="s",
                               num_cores=NC, num_subcores=NS)


@functools.partial(
    pl.kernel,
    out_type=jax.ShapeDtypeStruct((NC, B, SP, H), jnp.float32),
    mesh=_mesh,
    scratch_types=[
        pltpu.VMEM((CHUNK,), jnp.int32),
        pltpu.VMEM((CHUNK,), jnp.int32),
        pltpu.VMEM((CHUNK, H), jnp.float32),
        pltpu.VMEM((ZROWS, H), jnp.float32),
        pltpu.VMEM_SHARED((SP, H), jnp.float32),
        pltpu.SemaphoreType.DMA,
    ],
)
def _sc_feat_agg(gidx_hbm, sidx_hbm, table_hbm, out_hbm,
                 gidx_v, sidx_v, rows_v, zeros_v, acc, sem):
    cid = lax.axis_index("c")
    tid = lax.axis_index("s")
    zvec = jnp.zeros((16,), jnp.float32)
    for r in range(ZROWS):
        for c0 in range(H // 16):
            zeros_v[r, pl.ds(c0 * 16, 16)] = zvec
    for b in range(B):
        for j in range(RPT // ZROWS):
            pltpu.sync_copy(zeros_v,
                            acc.at[pl.ds(tid * RPT + j * ZROWS, ZROWS)])
        plsc.subcore_barrier()
        base0 = b * E + cid * (E // NC) + tid * PER_TILE

        def body(j, carry):
            base = base0 + j * CHUNK
            pltpu.sync_copy(gidx_hbm.at[pl.ds(base, CHUNK)], gidx_v)
            pltpu.sync_copy(sidx_hbm.at[pl.ds(base, CHUNK)], sidx_v)
            pltpu.async_copy(table_hbm.at[gidx_v], rows_v, sem).wait()
            pltpu.sync_copy(rows_v, acc.at[sidx_v], add=True)
            return carry

        lax.fori_loop(0, NCHUNK, body, 0)
        plsc.subcore_barrier()
        pltpu.sync_copy(acc.at[pl.ds(tid * RPT, RPT)],
                        out_hbm.at[cid, b, pl.ds(tid * RPT, RPT)])
        plsc.subcore_barrier()


@functools.partial(
    pl.kernel,
    out_type=jax.ShapeDtypeStruct((NC, 2, B, SP), jnp.float32),
    mesh=_mesh,
    scratch_types=[
        pltpu.VMEM((CHUNK,), jnp.int32),
        pltpu.VMEM((CHUNK,), jnp.int32),
        pltpu.VMEM((CHUNK,), jnp.float32),
        pltpu.VMEM((128,), jnp.float32),
        pltpu.VMEM_SHARED((SP,), jnp.float32),
        pltpu.VMEM_SHARED((SP,), jnp.float32),
    ],
)
def _sc_degrees(eidx_hbm, nidx_hbm, out_hbm,
                eidx_v, nidx_v, ones_v, zeros_v, dege, degn):
    cid = lax.axis_index("c")
    tid = lax.axis_index("s")
    zvec = jnp.zeros((16,), jnp.float32)
    ovec = jnp.ones((16,), jnp.float32)
    for i in range(128 // 16):
        zeros_v[pl.ds(i * 16, 16)] = zvec
    for i in range(CHUNK // 16):
        ones_v[pl.ds(i * 16, 16)] = ovec
    ones_v[pl.ds(CHUNK - 16, 16)] = ovec
    for b in range(B):
        for j in range(RPT // 128):
            pltpu.sync_copy(zeros_v, dege.at[pl.ds(tid * RPT + j * 128, 128)])
            pltpu.sync_copy(zeros_v, degn.at[pl.ds(tid * RPT + j * 128, 128)])
        plsc.subcore_barrier()
        base0 = b * E + cid * (E // NC) + tid * PER_TILE

        def body(j, carry):
            base = base0 + j * CHUNK
            pltpu.sync_copy(eidx_hbm.at[pl.ds(base, CHUNK)], eidx_v)
            pltpu.sync_copy(nidx_hbm.at[pl.ds(base, CHUNK)], nidx_v)
            pltpu.sync_copy(ones_v, dege.at[eidx_v], add=True)
            pltpu.sync_copy(ones_v, degn.at[nidx_v], add=True)
            return carry

        lax.fori_loop(0, NCHUNK, body, 0)
        plsc.subcore_barrier()
        pltpu.sync_copy(dege.at[pl.ds(tid * RPT, RPT)],
                        out_hbm.at[cid, 0, b, pl.ds(tid * RPT, RPT)])
        pltpu.sync_copy(degn.at[pl.ds(tid * RPT, RPT)],
                        out_hbm.at[cid, 1, b, pl.ds(tid * RPT, RPT)])
        plsc.subcore_barrier()


ENC_BLK = 400
MM_BLK = 1000
CMB_BLK = 1024


def _enc_body(x_ref, w_ref, o_ref):
    x = x_ref[...]
    w = w_ref[...]
    acc = jnp.zeros((ENC_BLK, H), jnp.float32)
    for t in range(T):
        ht = (x[:, 2 * t:2 * t + 1] * w[0:1, :]
              + x[:, 2 * t + 1:2 * t + 2] * w[1:2, :] + w[2:3, :])
        acc = acc + jnp.maximum(ht, 0.0)
    o_ref[...] = acc * (1.0 / T)


def _mm_body(x_ref, w_ref, o_ref):
    o_ref[...] = jnp.dot(x_ref[...], w_ref[...],
                         preferred_element_type=jnp.float32)


def _combine_body(p_ref, d_ref, o_ref):
    s = p_ref[0, 0] + p_ref[1, 0]
    d = d_ref[0, 0] + d_ref[1, 0]
    o_ref[0] = s / jnp.maximum(d, 1.0)


def _bn_body(p_ref, d_ref, gb_ref, o_ref):
    p = p_ref[0, 0] + p_ref[1, 0]
    d = d_ref[0, 0] + d_ref[1, 0]
    x = (p / jnp.maximum(d, 1.0))[:N]
    mean = jnp.mean(x, axis=0, keepdims=True)
    xc = x - mean
    var = jnp.mean(xc * xc, axis=0, keepdims=True)
    y = xc * lax.rsqrt(var + EPS) * gb_ref[0:1, :] + gb_ref[1:2, :]
    o_ref[0] = jnp.maximum(y, 0.0)


def _dec_body(h_ref, hs_ref, w_ref, b_ref, o_ref):
    o_ref[...] = (jnp.dot(h_ref[...] + hs_ref[...], w_ref[...],
                          preferred_element_type=jnp.float32) + b_ref[...])


def _encode(obs_flat, wenc8):
    return pl.pallas_call(
        _enc_body,
        grid=(B * N // ENC_BLK,),
        in_specs=[
            pl.BlockSpec((ENC_BLK, T * D_IN), lambda i: (i, 0)),
            pl.BlockSpec((8, H), lambda i: (0, 0)),
        ],
        out_specs=pl.BlockSpec((ENC_BLK, H), lambda i: (i, 0)),
        out_shape=jax.ShapeDtypeStruct((B * N, H), jnp.float32),
    )(obs_flat, wenc8)


def _matmul(x, w):
    return pl.pallas_call(
        _mm_body,
        grid=(B * N // MM_BLK,),
        in_specs=[
            pl.BlockSpec((MM_BLK, H), lambda i: (i, 0)),
            pl.BlockSpec((H, H), lambda i: (0, 0)),
        ],
        out_specs=pl.BlockSpec((MM_BLK, H), lambda i: (i, 0)),
        out_shape=jax.ShapeDtypeStruct((B * N, H), jnp.float32),
    )(x, w)


def _combine(parts, degp):
    return pl.pallas_call(
        _combine_body,
        grid=(B, SP // CMB_BLK),
        in_specs=[
            pl.BlockSpec((NC, 1, CMB_BLK, H), lambda b, i: (0, b, i, 0)),
            pl.BlockSpec((NC, 1, CMB_BLK, 1), lambda b, i: (0, b, i, 0)),
        ],
        out_specs=pl.BlockSpec((1, CMB_BLK, H), lambda b, i: (b, i, 0)),
        out_shape=jax.ShapeDtypeStruct((B, S, H), jnp.float32),
    )(parts, degp)


def _bn_relu(parts, degp, gb8):
    return pl.pallas_call(
        _bn_body,
        grid=(B,),
        in_specs=[
            pl.BlockSpec((NC, 1, SP, H), lambda b: (0, b, 0, 0)),
            pl.BlockSpec((NC, 1, SP, 1), lambda b: (0, b, 0, 0)),
            pl.BlockSpec((8, H), lambda b: (0, 0)),
        ],
        out_specs=pl.BlockSpec((1, N, H), lambda b: (b, 0, 0)),
        out_shape=jax.ShapeDtypeStruct((B, N, H), jnp.float32),
    )(parts, degp, gb8)


def _decode(h, hs, wdec, bdec):
    return pl.pallas_call(
        _dec_body,
        grid=(B * N // MM_BLK,),
        in_specs=[
            pl.BlockSpec((MM_BLK, H), lambda i: (i, 0)),
            pl.BlockSpec((MM_BLK, H), lambda i: (i, 0)),
            pl.BlockSpec((H, PRED * 2), lambda i: (0, 0)),
            pl.BlockSpec((1, PRED * 2), lambda i: (0, 0)),
        ],
        out_specs=pl.BlockSpec((MM_BLK, PRED * 2), lambda i: (i, 0)),
        out_shape=jax.ShapeDtypeStruct((B * N, PRED * 2), jnp.float32),
    )(h, hs, wdec, bdec)


def kernel(obs_traj, hyperedge_indices, W_enc, b_enc, theta0, theta1,
           gamma0, beta0, gamma1, beta1, W_dec, b_dec):
    hi = hyperedge_indices.astype(jnp.int32)
    node = hi[:, 0, :]
    edge = hi[:, 1, :]
    boff = jnp.arange(B, dtype=jnp.int32)[:, None] * S
    node_flat = node.reshape(-1)
    edge_flat = edge.reshape(-1)
    node_off = (node + boff).reshape(-1)
    edge_off = (edge + boff).reshape(-1)

    wenc8 = jnp.zeros((8, H), jnp.float32)
    wenc8 = wenc8.at[0:2].set(W_enc).at[2].set(b_enc)
    gb0 = jnp.zeros((8, H), jnp.float32).at[0].set(gamma0).at[1].set(beta0)
    gb1 = jnp.zeros((8, H), jnp.float32).at[0].set(gamma1).at[1].set(beta1)

    h = _encode(obs_traj.reshape(B * N, T * D_IN), wenc8)

    degp = _sc_degrees(edge_flat, node_flat)
    degp_e = degp[:, 0][..., None]
    degp_n = degp[:, 1][..., None]

    x = h
    for theta, gb in ((theta0, gb0), (theta1, gb1)):
        xt = _matmul(x, theta)
        eparts = _sc_feat_agg(node_off, edge_flat, xt)
        e_feat = _combine(eparts, degp_e)
        nparts = _sc_feat_agg(edge_off, node_flat,
                              e_feat.reshape(B * M, H))
        h_social = _bn_relu(nparts, degp_n, gb)
        x = h_social.reshape(B * N, H)

    out = _decode(h, x, W_dec, b_dec.reshape(1, PRED * 2))
    return out.reshape(B, N, PRED, 2)


# speedup vs baseline: 35.5608x; 3.2784x over previous
import functools

import jax
import jax.numpy as jnp
from jax import lax
from jax.experimental import pallas as pl
from jax.experimental.pallas import tpu as pltpu
from jax.experimental.pallas import tpu_sc as plsc

B, N, T, D_IN = 4, 10000, 8, 2
E = 160000
H = 128
PRED = 12
M = 10000
S = 10000
EPS = 1e-5

NC, NS = 2, 16
PER_TILE = E // (NC * NS)
CA = 96
NCH = PER_TILE // CA
TAIL = PER_TILE - NCH * CA
SP = 10240
RPT = SP // NS
ZROWS = 32

_mesh = plsc.VectorSubcoreMesh(core_axis_name="c", subcore_axis_name="s",
                               num_cores=NC, num_subcores=NS)


def _fill_idx(dst, src, off, n):
    for i in range(n // 16):
        dst[pl.ds(i * 16, 16)] = src[pl.ds(off + i * 16, 16)]


@functools.partial(
    pl.kernel,
    out_type=jax.ShapeDtypeStruct((NC, B, SP, H), jnp.float32),
    mesh=_mesh,
    scratch_types=[
        pltpu.VMEM((PER_TILE,), jnp.int32),
        pltpu.VMEM((PER_TILE,), jnp.int32),
        pltpu.VMEM((CA,), jnp.int32),
        pltpu.VMEM((CA,), jnp.int32),
        pltpu.VMEM((CA,), jnp.int32),
        pltpu.VMEM((CA,), jnp.int32),
        pltpu.VMEM((TAIL,), jnp.int32),
        pltpu.VMEM((TAIL,), jnp.int32),
        pltpu.VMEM((CA, H), jnp.float32),
        pltpu.VMEM((CA, H), jnp.float32),
        pltpu.VMEM((TAIL, H), jnp.float32),
        pltpu.VMEM((ZROWS, H), jnp.float32),
        pltpu.VMEM_SHARED((SP, H), jnp.float32),
        pltpu.SemaphoreType.DMA,
        pltpu.SemaphoreType.DMA,
        pltpu.SemaphoreType.DMA,
    ],
)
def _sc_feat_agg(gidx_hbm, sidx_hbm, table_hbm, out_hbm,
                 gbig, sbig, gs0, gs1, ss0, ss1, gst, sst,
                 rows0, rows1, rowst, zeros_v, acc, sem0, sem1, semt):
    cid = lax.axis_index("c")
    tid = lax.axis_index("s")
    gs = (gs0, gs1)
    ss = (ss0, ss1)
    rows = (rows0, rows1)
    sems = (sem0, sem1)
    zvec = jnp.zeros((16,), jnp.float32)
    for r in range(ZROWS):
        for c0 in range(H // 16):
            zeros_v[r, pl.ds(c0 * 16, 16)] = zvec
    for b in range(B):
        for j in range(RPT // ZROWS):
            pltpu.sync_copy(zeros_v,
                            acc.at[pl.ds(tid * RPT + j * ZROWS, ZROWS)])
        plsc.subcore_barrier()
        base0 = b * E + cid * (E // NC) + tid * PER_TILE
        pltpu.sync_copy(gidx_hbm.at[pl.ds(base0, PER_TILE)], gbig)
        pltpu.sync_copy(sidx_hbm.at[pl.ds(base0, PER_TILE)], sbig)
        pltpu.sync_copy(gidx_hbm.at[pl.ds(base0 + NCH * CA, TAIL)], gst)
        pltpu.sync_copy(sidx_hbm.at[pl.ds(base0 + NCH * CA, TAIL)], sst)
        for k in range(2):
            _fill_idx(gs[k], gbig, k * CA, CA)
            pltpu.async_copy(table_hbm.at[gs[k]], rows[k], sems[k])

        def body(p, carry):
            for k in range(2):
                j = 2 * p + k
                pltpu.make_async_copy(table_hbm.at[gs[k]], rows[k],
                                      sems[k]).wait()
                _fill_idx(ss[k], sbig, j * CA, CA)
                pltpu.sync_copy(rows[k], acc.at[ss[k]], add=True)

                @pl.when(j + 2 < NCH)
                def _():
                    _fill_idx(gs[k], gbig, (j + 2) * CA, CA)
                    pltpu.async_copy(table_hbm.at[gs[k]], rows[k], sems[k])
            return carry

        lax.fori_loop(0, NCH // 2, body, 0)
        pltpu.async_copy(table_hbm.at[gst], rowst, semt).wait()
        pltpu.sync_copy(rowst, acc.at[sst], add=True)
        plsc.subcore_barrier()
        pltpu.sync_copy(acc.at[pl.ds(tid * RPT, RPT)],
                        out_hbm.at[cid, b, pl.ds(tid * RPT, RPT)])
        plsc.subcore_barrier()


@functools.partial(
    pl.kernel,
    out_type=jax.ShapeDtypeStruct((NC, 2, B, SP), jnp.float32),
    mesh=_mesh,
    scratch_types=[
        pltpu.VMEM((PER_TILE,), jnp.int32),
        pltpu.VMEM((PER_TILE,), jnp.int32),
        pltpu.VMEM((CA,), jnp.int32),
        pltpu.VMEM((CA,), jnp.int32),
        pltpu.VMEM((TAIL,), jnp.int32),
        pltpu.VMEM((TAIL,), jnp.int32),
        pltpu.VMEM((CA,), jnp.float32),
        pltpu.VMEM((128,), jnp.float32),
        pltpu.VMEM_SHARED((SP,), jnp.float32),
        pltpu.VMEM_SHARED((SP,), jnp.float32),
    ],
)
def _sc_degrees(eidx_hbm, nidx_hbm, out_hbm,
                ebig, nbig, es, ns, est, nst, ones_v, zeros_v, dege, degn):
    cid = lax.axis_index("c")
    tid = lax.axis_index("s")
    zvec = jnp.zeros((16,), jnp.float32)
    ovec = jnp.ones((16,), jnp.float32)
    for i in range(128 // 16):
        zeros_v[pl.ds(i * 16, 16)] = zvec
    for i in range(CA // 16):
        ones_v[pl.ds(i * 16, 16)] = ovec
    for b in range(B):
        for j in range(RPT // 128):
            pltpu.sync_copy(zeros_v, dege.at[pl.ds(tid * RPT + j * 128, 128)])
            pltpu.sync_copy(zeros_v, degn.at[pl.ds(tid * RPT + j * 128, 128)])
        plsc.subcore_barrier()
        base0 = b * E + cid * (E // NC) + tid * PER_TILE
        pltpu.sync_copy(eidx_hbm.at[pl.ds(base0, PER_TILE)], ebig)
        pltpu.sync_copy(nidx_hbm.at[pl.ds(base0, PER_TILE)], nbig)
        pltpu.sync_copy(eidx_hbm.at[pl.ds(base0 + NCH * CA, TAIL)], est)
        pltpu.sync_copy(nidx_hbm.at[pl.ds(base0 + NCH * CA, TAIL)], nst)

        def body(j, carry):
            _fill_idx(es, ebig, j * CA, CA)
            _fill_idx(ns, nbig, j * CA, CA)
            pltpu.sync_copy(ones_v, dege.at[es], add=True)
            pltpu.sync_copy(ones_v, degn.at[ns], add=True)
            return carry

        lax.fori_loop(0, NCH, body, 0)
        pltpu.sync_copy(ones_v.at[pl.ds(0, TAIL)], dege.at[est], add=True)
        pltpu.sync_copy(ones_v.at[pl.ds(0, TAIL)], degn.at[nst], add=True)
        plsc.subcore_barrier()
        pltpu.sync_copy(dege.at[pl.ds(tid * RPT, RPT)],
                        out_hbm.at[cid, 0, b, pl.ds(tid * RPT, RPT)])
        pltpu.sync_copy(degn.at[pl.ds(tid * RPT, RPT)],
                        out_hbm.at[cid, 1, b, pl.ds(tid * RPT, RPT)])
        plsc.subcore_barrier()


ENC_BLK = 400
MM_BLK = 1000
CMB_BLK = 1024


def _enc_body(x_ref, w_ref, o_ref):
    x = x_ref[...]
    w = w_ref[...]
    acc = jnp.zeros((ENC_BLK, H), jnp.float32)
    for t in range(T):
        ht = (x[:, 2 * t:2 * t + 1] * w[0:1, :]
              + x[:, 2 * t + 1:2 * t + 2] * w[1:2, :] + w[2:3, :])
        acc = acc + jnp.maximum(ht, 0.0)
    o_ref[...] = acc * (1.0 / T)


def _mm_body(x_ref, w_ref, o_ref):
    o_ref[...] = jnp.dot(x_ref[...], w_ref[...],
                         preferred_element_type=jnp.float32)


def _combine_body(p_ref, d_ref, o_ref):
    s = p_ref[0, 0] + p_ref[1, 0]
    d = d_ref[0, 0] + d_ref[1, 0]
    o_ref[0] = s / jnp.maximum(d, 1.0)


def _bn_body(p_ref, d_ref, gb_ref, o_ref):
    p = p_ref[0, 0] + p_ref[1, 0]
    d = d_ref[0, 0] + d_ref[1, 0]
    x = (p / jnp.maximum(d, 1.0))[:N]
    mean = jnp.mean(x, axis=0, keepdims=True)
    xc = x - mean
    var = jnp.mean(xc * xc, axis=0, keepdims=True)
    y = xc * lax.rsqrt(var + EPS) * gb_ref[0:1, :] + gb_ref[1:2, :]
    o_ref[0] = jnp.maximum(y, 0.0)


def _dec_body(h_ref, hs_ref, w_ref, b_ref, o_ref):
    o_ref[...] = (jnp.dot(h_ref[...] + hs_ref[...], w_ref[...],
                          preferred_element_type=jnp.float32) + b_ref[...])


def _encode(obs_flat, wenc8):
    return pl.pallas_call(
        _enc_body,
        grid=(B * N // ENC_BLK,),
        in_specs=[
            pl.BlockSpec((ENC_BLK, T * D_IN), lambda i: (i, 0)),
            pl.BlockSpec((8, H), lambda i: (0, 0)),
        ],
        out_specs=pl.BlockSpec((ENC_BLK, H), lambda i: (i, 0)),
        out_shape=jax.ShapeDtypeStruct((B * N, H), jnp.float32),
    )(obs_flat, wenc8)


def _matmul(x, w):
    return pl.pallas_call(
        _mm_body,
        grid=(B * N // MM_BLK,),
        in_specs=[
            pl.BlockSpec((MM_BLK, H), lambda i: (i, 0)),
            pl.BlockSpec((H, H), lambda i: (0, 0)),
        ],
        out_specs=pl.BlockSpec((MM_BLK, H), lambda i: (i, 0)),
        out_shape=jax.ShapeDtypeStruct((B * N, H), jnp.float32),
    )(x, w)


def _combine(parts, degp):
    return pl.pallas_call(
        _combine_body,
        grid=(B, SP // CMB_BLK),
        in_specs=[
            pl.BlockSpec((NC, 1, CMB_BLK, H), lambda b, i: (0, b, i, 0)),
            pl.BlockSpec((NC, 1, CMB_BLK, 1), lambda b, i: (0, b, i, 0)),
        ],
        out_specs=pl.BlockSpec((1, CMB_BLK, H), lambda b, i: (b, i, 0)),
        out_shape=jax.ShapeDtypeStruct((B, S, H), jnp.float32),
    )(parts, degp)


def _bn_relu(parts, degp, gb8):
    return pl.pallas_call(
        _bn_body,
        grid=(B,),
        in_specs=[
            pl.BlockSpec((NC, 1, SP, H), lambda b: (0, b, 0, 0)),
            pl.BlockSpec((NC, 1, SP, 1), lambda b: (0, b, 0, 0)),
            pl.BlockSpec((8, H), lambda b: (0, 0)),
        ],
        out_specs=pl.BlockSpec((1, N, H), lambda b: (b, 0, 0)),
        out_shape=jax.ShapeDtypeStruct((B, N, H), jnp.float32),
    )(parts, degp, gb8)


def _decode(h, hs, wdec, bdec):
    return pl.pallas_call(
        _dec_body,
        grid=(B * N // MM_BLK,),
        in_specs=[
            pl.BlockSpec((MM_BLK, H), lambda i: (i, 0)),
            pl.BlockSpec((MM_BLK, H), lambda i: (i, 0)),
            pl.BlockSpec((H, PRED * 2), lambda i: (0, 0)),
            pl.BlockSpec((1, PRED * 2), lambda i: (0, 0)),
        ],
        out_specs=pl.BlockSpec((MM_BLK, PRED * 2), lambda i: (i, 0)),
        out_shape=jax.ShapeDtypeStruct((B * N, PRED * 2), jnp.float32),
    )(h, hs, wdec, bdec)


def kernel(obs_traj, hyperedge_indices, W_enc, b_enc, theta0, theta1,
           gamma0, beta0, gamma1, beta1, W_dec, b_dec):
    hi = hyperedge_indices.astype(jnp.int32)
    node = hi[:, 0, :]
    edge = hi[:, 1, :]
    boff = jnp.arange(B, dtype=jnp.int32)[:, None] * S
    node_flat = node.reshape(-1)
    edge_flat = edge.reshape(-1)
    node_off = (node + boff).reshape(-1)
    edge_off = (edge + boff).reshape(-1)

    wenc8 = jnp.zeros((8, H), jnp.float32)
    wenc8 = wenc8.at[0:2].set(W_enc).at[2].set(b_enc)
    gb0 = jnp.zeros((8, H), jnp.float32).at[0].set(gamma0).at[1].set(beta0)
    gb1 = jnp.zeros((8, H), jnp.float32).at[0].set(gamma1).at[1].set(beta1)

    h = _encode(obs_traj.reshape(B * N, T * D_IN), wenc8)

    degp = _sc_degrees(edge_flat, node_flat)
    degp_e = degp[:, 0][..., None]
    degp_n = degp[:, 1][..., None]

    x = h
    for theta, gb in ((theta0, gb0), (theta1, gb1)):
        xt = _matmul(x, theta)
        eparts = _sc_feat_agg(node_off, edge_flat, xt)
        e_feat = _combine(eparts, degp_e)
        nparts = _sc_feat_agg(edge_off, node_flat,
                              e_feat.reshape(B * M, H))
        h_social = _bn_relu(nparts, degp_n, gb)
        x = h_social.reshape(B * N, H)

    out = _decode(h, x, W_dec, b_dec.reshape(1, PRED * 2))
    return out.reshape(B, N, PRED, 2)


# speedup vs baseline: 36.0261x; 1.0131x over previous
import functools

import jax
import jax.numpy as jnp
from jax import lax
from jax.experimental import pallas as pl
from jax.experimental.pallas import tpu as pltpu
from jax.experimental.pallas import tpu_sc as plsc

B, N, T, D_IN = 4, 10000, 8, 2
E = 160000
H = 128
PRED = 12
M = 10000
S = 10000
EPS = 1e-5

NC, NS = 2, 16
PER_TILE = E // (NC * NS)
CA = 96
NCH = PER_TILE // CA
TAIL = PER_TILE - NCH * CA
SP = 10240
RPT = SP // NS
ZROWS = 32

_mesh = plsc.VectorSubcoreMesh(core_axis_name="c", subcore_axis_name="s",
                               num_cores=NC, num_subcores=NS)


def _fill_idx(dst, src, off, n):
    for i in range(n // 16):
        dst[pl.ds(i * 16, 16)] = src[pl.ds(off + i * 16, 16)]


@functools.partial(
    pl.kernel,
    out_type=jax.ShapeDtypeStruct((NC, B, SP, H), jnp.float32),
    mesh=_mesh,
    scratch_types=[
        pltpu.VMEM((PER_TILE,), jnp.int32),
        pltpu.VMEM((PER_TILE,), jnp.int32),
        pltpu.VMEM((CA,), jnp.int32),
        pltpu.VMEM((CA,), jnp.int32),
        pltpu.VMEM((CA,), jnp.int32),
        pltpu.VMEM((CA,), jnp.int32),
        pltpu.VMEM((TAIL,), jnp.int32),
        pltpu.VMEM((TAIL,), jnp.int32),
        pltpu.VMEM((CA, H), jnp.float32),
        pltpu.VMEM((CA, H), jnp.float32),
        pltpu.VMEM((TAIL, H), jnp.float32),
        pltpu.VMEM((ZROWS, H), jnp.float32),
        pltpu.VMEM_SHARED((SP, H), jnp.float32),
        pltpu.SemaphoreType.DMA,
        pltpu.SemaphoreType.DMA,
        pltpu.SemaphoreType.DMA,
    ],
)
def _sc_feat_agg(gidx_hbm, sidx_hbm, table_hbm, out_hbm,
                 gbig, sbig, gs0, gs1, ss0, ss1, gst, sst,
                 rows0, rows1, rowst, zeros_v, acc, sem0, sem1, semt):
    cid = lax.axis_index("c")
    tid = lax.axis_index("s")
    gs = (gs0, gs1)
    ss = (ss0, ss1)
    rows = (rows0, rows1)
    sems = (sem0, sem1)
    zvec = jnp.zeros((16,), jnp.float32)
    for r in range(ZROWS):
        for c0 in range(H // 16):
            zeros_v[r, pl.ds(c0 * 16, 16)] = zvec
    for b in range(B):
        for j in range(RPT // ZROWS):
            pltpu.sync_copy(zeros_v,
                            acc.at[pl.ds(tid * RPT + j * ZROWS, ZROWS)])
        plsc.subcore_barrier()
        base0 = b * E + cid * (E // NC) + tid * PER_TILE
        pltpu.sync_copy(gidx_hbm.at[pl.ds(base0, PER_TILE)], gbig)
        pltpu.sync_copy(sidx_hbm.at[pl.ds(base0, PER_TILE)], sbig)
        pltpu.sync_copy(gidx_hbm.at[pl.ds(base0 + NCH * CA, TAIL)], gst)
        pltpu.sync_copy(sidx_hbm.at[pl.ds(base0 + NCH * CA, TAIL)], sst)
        for k in range(2):
            _fill_idx(gs[k], gbig, k * CA, CA)
            pltpu.async_copy(table_hbm.at[gs[k]], rows[k], sems[k])

        def body(p, carry):
            for k in range(2):
                j = 2 * p + k
                pltpu.make_async_copy(table_hbm.at[gs[k]], rows[k],
                                      sems[k]).wait()
                _fill_idx(ss[k], sbig, j * CA, CA)
                pltpu.sync_copy(rows[k], acc.at[ss[k]], add=True)

                @pl.when(j + 2 < NCH)
                def _():
                    _fill_idx(gs[k], gbig, (j + 2) * CA, CA)
                    pltpu.async_copy(table_hbm.at[gs[k]], rows[k], sems[k])
            return carry

        lax.fori_loop(0, NCH // 2, body, 0)
        pltpu.async_copy(table_hbm.at[gst], rowst, semt).wait()
        pltpu.sync_copy(rowst, acc.at[sst], add=True)
        plsc.subcore_barrier()
        pltpu.sync_copy(acc.at[pl.ds(tid * RPT, RPT)],
                        out_hbm.at[cid, b, pl.ds(tid * RPT, RPT)])
        plsc.subcore_barrier()


@functools.partial(
    pl.kernel,
    out_type=jax.ShapeDtypeStruct((NC, 2, B, SP), jnp.float32),
    mesh=_mesh,
    scratch_types=[
        pltpu.VMEM((PER_TILE,), jnp.int32),
        pltpu.VMEM((PER_TILE,), jnp.int32),
        pltpu.VMEM((CA,), jnp.int32),
        pltpu.VMEM((CA,), jnp.int32),
        pltpu.VMEM((TAIL,), jnp.int32),
        pltpu.VMEM((TAIL,), jnp.int32),
        pltpu.VMEM((CA,), jnp.float32),
        pltpu.VMEM((128,), jnp.float32),
        pltpu.VMEM_SHARED((SP,), jnp.float32),
        pltpu.VMEM_SHARED((SP,), jnp.float32),
    ],
)
def _sc_degrees(eidx_hbm, nidx_hbm, out_hbm,
                ebig, nbig, es, ns, est, nst, ones_v, zeros_v, dege, degn):
    cid = lax.axis_index("c")
    tid = lax.axis_index("s")
    zvec = jnp.zeros((16,), jnp.float32)
    ovec = jnp.ones((16,), jnp.float32)
    for i in range(128 // 16):
        zeros_v[pl.ds(i * 16, 16)] = zvec
    for i in range(CA // 16):
        ones_v[pl.ds(i * 16, 16)] = ovec
    for b in range(B):
        for j in range(RPT // 128):
            pltpu.sync_copy(zeros_v, dege.at[pl.ds(tid * RPT + j * 128, 128)])
            pltpu.sync_copy(zeros_v, degn.at[pl.ds(tid * RPT + j * 128, 128)])
        plsc.subcore_barrier()
        base0 = b * E + cid * (E // NC) + tid * PER_TILE
        pltpu.sync_copy(eidx_hbm.at[pl.ds(base0, PER_TILE)], ebig)
        pltpu.sync_copy(nidx_hbm.at[pl.ds(base0, PER_TILE)], nbig)
        pltpu.sync_copy(eidx_hbm.at[pl.ds(base0 + NCH * CA, TAIL)], est)
        pltpu.sync_copy(nidx_hbm.at[pl.ds(base0 + NCH * CA, TAIL)], nst)

        def body(j, carry):
            _fill_idx(es, ebig, j * CA, CA)
            _fill_idx(ns, nbig, j * CA, CA)
            pltpu.sync_copy(ones_v, dege.at[es], add=True)
            pltpu.sync_copy(ones_v, degn.at[ns], add=True)
            return carry

        lax.fori_loop(0, NCH, body, 0)
        pltpu.sync_copy(ones_v.at[pl.ds(0, TAIL)], dege.at[est], add=True)
        pltpu.sync_copy(ones_v.at[pl.ds(0, TAIL)], degn.at[nst], add=True)
        plsc.subcore_barrier()
        pltpu.sync_copy(dege.at[pl.ds(tid * RPT, RPT)],
                        out_hbm.at[cid, 0, b, pl.ds(tid * RPT, RPT)])
        pltpu.sync_copy(degn.at[pl.ds(tid * RPT, RPT)],
                        out_hbm.at[cid, 1, b, pl.ds(tid * RPT, RPT)])
        plsc.subcore_barrier()


ENC_BLK = 400
MM_BLK = 1000
CMB_BLK = 1024


def _enc_body(x_ref, w_ref, t_ref, o_ref, xt_ref):
    x = x_ref[...]
    w = w_ref[...]
    acc = jnp.zeros((ENC_BLK, H), jnp.float32)
    for t in range(T):
        ht = (x[:, 2 * t:2 * t + 1] * w[0:1, :]
              + x[:, 2 * t + 1:2 * t + 2] * w[1:2, :] + w[2:3, :])
        acc = acc + jnp.maximum(ht, 0.0)
    h = acc * (1.0 / T)
    o_ref[...] = h
    xt_ref[...] = jnp.dot(h, t_ref[...], preferred_element_type=jnp.float32)


def _mm_body(x_ref, w_ref, o_ref):
    o_ref[...] = jnp.dot(x_ref[...], w_ref[...],
                         preferred_element_type=jnp.float32)


def _combine_body(p_ref, d_ref, o_ref):
    s = p_ref[0, 0] + p_ref[1, 0]
    d = d_ref[0, 0] + d_ref[1, 0]
    o_ref[0] = s / jnp.maximum(d, 1.0)


def _bn_body(p_ref, d_ref, gb_ref, o_ref):
    p = p_ref[0, 0] + p_ref[1, 0]
    d = d_ref[0, 0] + d_ref[1, 0]
    x = (p / jnp.maximum(d, 1.0))[:N]
    mean = jnp.mean(x, axis=0, keepdims=True)
    xc = x - mean
    var = jnp.mean(xc * xc, axis=0, keepdims=True)
    y = xc * lax.rsqrt(var + EPS) * gb_ref[0:1, :] + gb_ref[1:2, :]
    o_ref[0] = jnp.maximum(y, 0.0)


def _dec_body(h_ref, hs_ref, w_ref, b_ref, o_ref):
    o_ref[...] = (jnp.dot(h_ref[...] + hs_ref[...], w_ref[...],
                          preferred_element_type=jnp.float32) + b_ref[...])


def _encode(obs_flat, wenc8, theta):
    return pl.pallas_call(
        _enc_body,
        grid=(B * N // ENC_BLK,),
        in_specs=[
            pl.BlockSpec((ENC_BLK, T * D_IN), lambda i: (i, 0)),
            pl.BlockSpec((8, H), lambda i: (0, 0)),
            pl.BlockSpec((H, H), lambda i: (0, 0)),
        ],
        out_specs=[
            pl.BlockSpec((ENC_BLK, H), lambda i: (i, 0)),
            pl.BlockSpec((ENC_BLK, H), lambda i: (i, 0)),
        ],
        out_shape=[
            jax.ShapeDtypeStruct((B * N, H), jnp.float32),
            jax.ShapeDtypeStruct((B * N, H), jnp.float32),
        ],
    )(obs_flat, wenc8, theta)


def _matmul(x, w):
    return pl.pallas_call(
        _mm_body,
        grid=(B * N // MM_BLK,),
        in_specs=[
            pl.BlockSpec((MM_BLK, H), lambda i: (i, 0)),
            pl.BlockSpec((H, H), lambda i: (0, 0)),
        ],
        out_specs=pl.BlockSpec((MM_BLK, H), lambda i: (i, 0)),
        out_shape=jax.ShapeDtypeStruct((B * N, H), jnp.float32),
    )(x, w)


def _combine(parts, degp):
    return pl.pallas_call(
        _combine_body,
        grid=(B, SP // CMB_BLK),
        in_specs=[
            pl.BlockSpec((NC, 1, CMB_BLK, H), lambda b, i: (0, b, i, 0)),
            pl.BlockSpec((NC, 1, CMB_BLK, 1), lambda b, i: (0, b, i, 0)),
        ],
        out_specs=pl.BlockSpec((1, CMB_BLK, H), lambda b, i: (b, i, 0)),
        out_shape=jax.ShapeDtypeStruct((B, S, H), jnp.float32),
    )(parts, degp)


def _bn_relu(parts, degp, gb8):
    return pl.pallas_call(
        _bn_body,
        grid=(B,),
        in_specs=[
            pl.BlockSpec((NC, 1, SP, H), lambda b: (0, b, 0, 0)),
            pl.BlockSpec((NC, 1, SP, 1), lambda b: (0, b, 0, 0)),
            pl.BlockSpec((8, H), lambda b: (0, 0)),
        ],
        out_specs=pl.BlockSpec((1, N, H), lambda b: (b, 0, 0)),
        out_shape=jax.ShapeDtypeStruct((B, N, H), jnp.float32),
    )(parts, degp, gb8)


def _decode(h, hs, wdec, bdec):
    return pl.pallas_call(
        _dec_body,
        grid=(B * N // MM_BLK,),
        in_specs=[
            pl.BlockSpec((MM_BLK, H), lambda i: (i, 0)),
            pl.BlockSpec((MM_BLK, H), lambda i: (i, 0)),
            pl.BlockSpec((H, PRED * 2), lambda i: (0, 0)),
            pl.BlockSpec((1, PRED * 2), lambda i: (0, 0)),
        ],
        out_specs=pl.BlockSpec((MM_BLK, PRED * 2), lambda i: (i, 0)),
        out_shape=jax.ShapeDtypeStruct((B * N, PRED * 2), jnp.float32),
    )(h, hs, wdec, bdec)


def kernel(obs_traj, hyperedge_indices, W_enc, b_enc, theta0, theta1,
           gamma0, beta0, gamma1, beta1, W_dec, b_dec):
    hi = hyperedge_indices.astype(jnp.int32)
    node = hi[:, 0, :]
    edge = hi[:, 1, :]
    boff = jnp.arange(B, dtype=jnp.int32)[:, None] * S
    node_flat = node.reshape(-1)
    edge_flat = edge.reshape(-1)
    node_off = (node + boff).reshape(-1)
    edge_off = (edge + boff).reshape(-1)

    wenc8 = jnp.zeros((8, H), jnp.float32)
    wenc8 = wenc8.at[0:2].set(W_enc).at[2].set(b_enc)
    gb0 = jnp.zeros((8, H), jnp.float32).at[0].set(gamma0).at[1].set(beta0)
    gb1 = jnp.zeros((8, H), jnp.float32).at[0].set(gamma1).at[1].set(beta1)

    degp = _sc_degrees(edge_flat, node_flat)
    degp_e = degp[:, 0][..., None]
    degp_n = degp[:, 1][..., None]

    h, xt0 = _encode(obs_traj.reshape(B * N, T * D_IN), wenc8, theta0)

    x = h
    for li, (theta, gb) in enumerate(((theta0, gb0), (theta1, gb1))):
        xt = xt0 if li == 0 else _matmul(x, theta)
        eparts = _sc_feat_agg(node_off, edge_flat, xt)
        e_feat = _combine(eparts, degp_e)
        nparts = _sc_feat_agg(edge_off, node_flat,
                              e_feat.reshape(B * M, H))
        h_social = _bn_relu(nparts, degp_n, gb)
        x = h_social.reshape(B * N, H)

    out = _decode(h, x, W_dec, b_dec.reshape(1, PRED * 2))
    return out.reshape(B, N, PRED, 2)


# speedup vs baseline: 36.8632x; 1.0232x over previous
import functools

import jax
import jax.numpy as jnp
from jax import lax
from jax.experimental import pallas as pl
from jax.experimental.pallas import tpu as pltpu
from jax.experimental.pallas import tpu_sc as plsc

B, N, T, D_IN = 4, 10000, 8, 2
E = 160000
H = 128
PRED = 12
M = 10000
S = 10000
EPS = 1e-5

NC, NS = 2, 16
PER_TILE = E // (NC * NS)
CA = 96
NCH = PER_TILE // CA
TAIL = PER_TILE - NCH * CA
SP = 10240
RPT = SP // NS
ZROWS = 32

_mesh = plsc.VectorSubcoreMesh(core_axis_name="c", subcore_axis_name="s",
                               num_cores=NC, num_subcores=NS)


def _fill_idx(dst, src, off, n):
    for i in range(n // 16):
        dst[pl.ds(i * 16, 16)] = src[pl.ds(off + i * 16, 16)]


@functools.partial(
    pl.kernel,
    out_type=jax.ShapeDtypeStruct((NC, B, SP, H), jnp.float32),
    mesh=_mesh,
    scratch_types=[
        pltpu.VMEM((PER_TILE,), jnp.int32),
        pltpu.VMEM((PER_TILE,), jnp.int32),
        pltpu.VMEM((CA,), jnp.int32),
        pltpu.VMEM((CA,), jnp.int32),
        pltpu.VMEM((CA,), jnp.int32),
        pltpu.VMEM((CA,), jnp.int32),
        pltpu.VMEM((TAIL,), jnp.int32),
        pltpu.VMEM((TAIL,), jnp.int32),
        pltpu.VMEM((CA, H), jnp.float32),
        pltpu.VMEM((CA, H), jnp.float32),
        pltpu.VMEM((TAIL, H), jnp.float32),
        pltpu.VMEM((ZROWS, H), jnp.float32),
        pltpu.VMEM_SHARED((SP, H), jnp.float32),
        pltpu.SemaphoreType.DMA,
        pltpu.SemaphoreType.DMA,
        pltpu.SemaphoreType.DMA,
    ],
)
def _sc_feat_agg(gidx_hbm, sidx_hbm, table_hbm, out_hbm,
                 gbig, sbig, gs0, gs1, ss0, ss1, gst, sst,
                 rows0, rows1, rowst, zeros_v, acc, sem0, sem1, semt):
    cid = lax.axis_index("c")
    tid = lax.axis_index("s")
    gs = (gs0, gs1)
    ss = (ss0, ss1)
    rows = (rows0, rows1)
    sems = (sem0, sem1)
    zvec = jnp.zeros((16,), jnp.float32)
    for r in range(ZROWS):
        for c0 in range(H // 16):
            zeros_v[r, pl.ds(c0 * 16, 16)] = zvec
    for b in range(B):
        for j in range(RPT // ZROWS):
            pltpu.sync_copy(zeros_v,
                            acc.at[pl.ds(tid * RPT + j * ZROWS, ZROWS)])
        plsc.subcore_barrier()
        base0 = b * E + cid * (E // NC) + tid * PER_TILE
        pltpu.sync_copy(gidx_hbm.at[pl.ds(base0, PER_TILE)], gbig)
        pltpu.sync_copy(sidx_hbm.at[pl.ds(base0, PER_TILE)], sbig)
        pltpu.sync_copy(gidx_hbm.at[pl.ds(base0 + NCH * CA, TAIL)], gst)
        pltpu.sync_copy(sidx_hbm.at[pl.ds(base0 + NCH * CA, TAIL)], sst)
        for k in range(2):
            _fill_idx(gs[k], gbig, k * CA, CA)
            pltpu.async_copy(table_hbm.at[gs[k]], rows[k], sems[k])

        def body(p, carry):
            for k in range(2):
                j = 2 * p + k
                pltpu.make_async_copy(table_hbm.at[gs[k]], rows[k],
                                      sems[k]).wait()
                _fill_idx(ss[k], sbig, j * CA, CA)
                pltpu.sync_copy(rows[k], acc.at[ss[k]], add=True)

                @pl.when(j + 2 < NCH)
                def _():
                    _fill_idx(gs[k], gbig, (j + 2) * CA, CA)
                    pltpu.async_copy(table_hbm.at[gs[k]], rows[k], sems[k])
            return carry

        lax.fori_loop(0, NCH // 2, body, 0)
        pltpu.async_copy(table_hbm.at[gst], rowst, semt).wait()
        pltpu.sync_copy(rowst, acc.at[sst], add=True)
        plsc.subcore_barrier()
        pltpu.sync_copy(acc.at[pl.ds(tid * RPT, RPT)],
                        out_hbm.at[cid, b, pl.ds(tid * RPT, RPT)])
        plsc.subcore_barrier()


@functools.partial(
    pl.kernel,
    out_type=jax.ShapeDtypeStruct((NC, 2, B, SP), jnp.float32),
    mesh=_mesh,
    scratch_types=[
        pltpu.VMEM((PER_TILE,), jnp.int32),
        pltpu.VMEM((PER_TILE,), jnp.int32),
        pltpu.VMEM((CA,), jnp.int32),
        pltpu.VMEM((CA,), jnp.int32),
        pltpu.VMEM((TAIL,), jnp.int32),
        pltpu.VMEM((TAIL,), jnp.int32),
        pltpu.VMEM((CA,), jnp.float32),
        pltpu.VMEM((128,), jnp.float32),
        pltpu.VMEM_SHARED((SP,), jnp.float32),
        pltpu.VMEM_SHARED((SP,), jnp.float32),
    ],
)
def _sc_degrees(eidx_hbm, nidx_hbm, out_hbm,
                ebig, nbig, es, ns, est, nst, ones_v, zeros_v, dege, degn):
    cid = lax.axis_index("c")
    tid = lax.axis_index("s")
    zvec = jnp.zeros((16,), jnp.float32)
    ovec = jnp.ones((16,), jnp.float32)
    for i in range(128 // 16):
        zeros_v[pl.ds(i * 16, 16)] = zvec
    for i in range(CA // 16):
        ones_v[pl.ds(i * 16, 16)] = ovec
    for b in range(B):
        for j in range(RPT // 128):
            pltpu.sync_copy(zeros_v, dege.at[pl.ds(tid * RPT + j * 128, 128)])
            pltpu.sync_copy(zeros_v, degn.at[pl.ds(tid * RPT + j * 128, 128)])
        plsc.subcore_barrier()
        base0 = b * E + cid * (E // NC) + tid * PER_TILE
        pltpu.sync_copy(eidx_hbm.at[pl.ds(base0, PER_TILE)], ebig)
        pltpu.sync_copy(nidx_hbm.at[pl.ds(base0, PER_TILE)], nbig)
        pltpu.sync_copy(eidx_hbm.at[pl.ds(base0 + NCH * CA, TAIL)], est)
        pltpu.sync_copy(nidx_hbm.at[pl.ds(base0 + NCH * CA, TAIL)], nst)

        def body(j, carry):
            _fill_idx(es, ebig, j * CA, CA)
            _fill_idx(ns, nbig, j * CA, CA)
            pltpu.sync_copy(ones_v, dege.at[es], add=True)
            pltpu.sync_copy(ones_v, degn.at[ns], add=True)
            return carry

        lax.fori_loop(0, NCH, body, 0)
        pltpu.sync_copy(ones_v.at[pl.ds(0, TAIL)], dege.at[est], add=True)
        pltpu.sync_copy(ones_v.at[pl.ds(0, TAIL)], degn.at[nst], add=True)
        plsc.subcore_barrier()
        pltpu.sync_copy(dege.at[pl.ds(tid * RPT, RPT)],
                        out_hbm.at[cid, 0, b, pl.ds(tid * RPT, RPT)])
        pltpu.sync_copy(degn.at[pl.ds(tid * RPT, RPT)],
                        out_hbm.at[cid, 1, b, pl.ds(tid * RPT, RPT)])
        plsc.subcore_barrier()


ENC_BLK = 400
MM_BLK = 1000
CMB_BLK = 1024


def _enc_body(x_ref, w_ref, t_ref, o_ref, xt_ref):
    x = x_ref[...]
    w = w_ref[...]
    acc = jnp.zeros((ENC_BLK, H), jnp.float32)
    for t in range(T):
        ht = (x[:, 2 * t:2 * t + 1] * w[0:1, :]
              + x[:, 2 * t + 1:2 * t + 2] * w[1:2, :] + w[2:3, :])
        acc = acc + jnp.maximum(ht, 0.0)
    h = acc * (1.0 / T)
    o_ref[...] = h
    xt_ref[...] = jnp.dot(h, t_ref[...], preferred_element_type=jnp.float32)


def _mm_body(x_ref, w_ref, o_ref):
    o_ref[...] = jnp.dot(x_ref[...], w_ref[...],
                         preferred_element_type=jnp.float32)


def _combine_body(p_ref, d_ref, o_ref):
    s = p_ref[0, 0] + p_ref[1, 0]
    d = d_ref[0, 0] + d_ref[1, 0]
    o_ref[0] = s / jnp.maximum(d, 1.0)


def _bn_core(p_ref, d_ref, gb_ref):
    p = p_ref[0, 0] + p_ref[1, 0]
    d = d_ref[0, 0] + d_ref[1, 0]
    x = (p / jnp.maximum(d, 1.0))[:N]
    mean = jnp.mean(x, axis=0, keepdims=True)
    xc = x - mean
    var = jnp.mean(xc * xc, axis=0, keepdims=True)
    y = xc * lax.rsqrt(var + EPS) * gb_ref[0:1, :] + gb_ref[1:2, :]
    return jnp.maximum(y, 0.0)


def _bn_body(p_ref, d_ref, gb_ref, o_ref):
    o_ref[0] = _bn_core(p_ref, d_ref, gb_ref)


def _bn_mm_body(p_ref, d_ref, gb_ref, t_ref, xt_ref):
    hs = _bn_core(p_ref, d_ref, gb_ref)
    xt_ref[0] = jnp.dot(hs, t_ref[...], preferred_element_type=jnp.float32)


def _dec_body(h_ref, hs_ref, w_ref, b_ref, o_ref):
    o_ref[...] = (jnp.dot(h_ref[...] + hs_ref[...], w_ref[...],
                          preferred_element_type=jnp.float32) + b_ref[...])


def _encode(obs_flat, wenc8, theta):
    return pl.pallas_call(
        _enc_body,
        grid=(B * N // ENC_BLK,),
        in_specs=[
            pl.BlockSpec((ENC_BLK, T * D_IN), lambda i: (i, 0)),
            pl.BlockSpec((8, H), lambda i: (0, 0)),
            pl.BlockSpec((H, H), lambda i: (0, 0)),
        ],
        out_specs=[
            pl.BlockSpec((ENC_BLK, H), lambda i: (i, 0)),
            pl.BlockSpec((ENC_BLK, H), lambda i: (i, 0)),
        ],
        out_shape=[
            jax.ShapeDtypeStruct((B * N, H), jnp.float32),
            jax.ShapeDtypeStruct((B * N, H), jnp.float32),
        ],
    )(obs_flat, wenc8, theta)


def _matmul(x, w):
    return pl.pallas_call(
        _mm_body,
        grid=(B * N // MM_BLK,),
        in_specs=[
            pl.BlockSpec((MM_BLK, H), lambda i: (i, 0)),
            pl.BlockSpec((H, H), lambda i: (0, 0)),
        ],
        out_specs=pl.BlockSpec((MM_BLK, H), lambda i: (i, 0)),
        out_shape=jax.ShapeDtypeStruct((B * N, H), jnp.float32),
    )(x, w)


def _combine(parts, degp):
    return pl.pallas_call(
        _combine_body,
        grid=(B, SP // CMB_BLK),
        in_specs=[
            pl.BlockSpec((NC, 1, CMB_BLK, H), lambda b, i: (0, b, i, 0)),
            pl.BlockSpec((NC, 1, CMB_BLK, 1), lambda b, i: (0, b, i, 0)),
        ],
        out_specs=pl.BlockSpec((1, CMB_BLK, H), lambda b, i: (b, i, 0)),
        out_shape=jax.ShapeDtypeStruct((B, S, H), jnp.float32),
    )(parts, degp)


def _bn_relu(parts, degp, gb8):
    return pl.pallas_call(
        _bn_body,
        grid=(B,),
        in_specs=[
            pl.BlockSpec((NC, 1, SP, H), lambda b: (0, b, 0, 0)),
            pl.BlockSpec((NC, 1, SP, 1), lambda b: (0, b, 0, 0)),
            pl.BlockSpec((8, H), lambda b: (0, 0)),
        ],
        out_specs=pl.BlockSpec((1, N, H), lambda b: (b, 0, 0)),
        out_shape=jax.ShapeDtypeStruct((B, N, H), jnp.float32),
    )(parts, degp, gb8)


def _bn_relu_mm(parts, degp, gb8, theta):
    return pl.pallas_call(
        _bn_mm_body,
        grid=(B,),
        in_specs=[
            pl.BlockSpec((NC, 1, SP, H), lambda b: (0, b, 0, 0)),
            pl.BlockSpec((NC, 1, SP, 1), lambda b: (0, b, 0, 0)),
            pl.BlockSpec((8, H), lambda b: (0, 0)),
            pl.BlockSpec((H, H), lambda b: (0, 0)),
        ],
        out_specs=pl.BlockSpec((1, N, H), lambda b: (b, 0, 0)),
        out_shape=jax.ShapeDtypeStruct((B, N, H), jnp.float32),
    )(parts, degp, gb8, theta)


def _decode(h, hs, wdec, bdec):
    return pl.pallas_call(
        _dec_body,
        grid=(B * N // MM_BLK,),
        in_specs=[
            pl.BlockSpec((MM_BLK, H), lambda i: (i, 0)),
            pl.BlockSpec((MM_BLK, H), lambda i: (i, 0)),
            pl.BlockSpec((H, PRED * 2), lambda i: (0, 0)),
            pl.BlockSpec((1, PRED * 2), lambda i: (0, 0)),
        ],
        out_specs=pl.BlockSpec((MM_BLK, PRED * 2), lambda i: (i, 0)),
        out_shape=jax.ShapeDtypeStruct((B * N, PRED * 2), jnp.float32),
    )(h, hs, wdec, bdec)


def kernel(obs_traj, hyperedge_indices, W_enc, b_enc, theta0, theta1,
           gamma0, beta0, gamma1, beta1, W_dec, b_dec):
    hi = hyperedge_indices.astype(jnp.int32)
    node = hi[:, 0, :]
    edge = hi[:, 1, :]
    boff = jnp.arange(B, dtype=jnp.int32)[:, None] * S
    node_flat = node.reshape(-1)
    edge_flat = edge.reshape(-1)
    node_off = (node + boff).reshape(-1)
    edge_off = (edge + boff).reshape(-1)

    wenc8 = jnp.zeros((8, H), jnp.float32)
    wenc8 = wenc8.at[0:2].set(W_enc).at[2].set(b_enc)
    gb0 = jnp.zeros((8, H), jnp.float32).at[0].set(gamma0).at[1].set(beta0)
    gb1 = jnp.zeros((8, H), jnp.float32).at[0].set(gamma1).at[1].set(beta1)

    degp = _sc_degrees(edge_flat, node_flat)
    degp_e = degp[:, 0][..., None]
    degp_n = degp[:, 1][..., None]

    h, xt0 = _encode(obs_traj.reshape(B * N, T * D_IN), wenc8, theta0)

    eparts = _sc_feat_agg(node_off, edge_flat, xt0)
    e_feat = _combine(eparts, degp_e)
    nparts = _sc_feat_agg(edge_off, node_flat, e_feat.reshape(B * M, H))
    xt1 = _bn_relu_mm(nparts, degp_n, gb0, theta1)

    eparts = _sc_feat_agg(node_off, edge_flat, xt1.reshape(B * N, H))
    e_feat = _combine(eparts, degp_e)
    nparts = _sc_feat_agg(edge_off, node_flat, e_feat.reshape(B * M, H))
    h_social = _bn_relu(nparts, degp_n, gb1)

    out = _decode(h, h_social.reshape(B * N, H), W_dec,
                  b_dec.reshape(1, PRED * 2))
    return out.reshape(B, N, PRED, 2)
